# baseline (speedup 1.0000x reference)
import functools

import jax
import jax.numpy as jnp
from jax import lax
from jax.experimental import pallas as pl
from jax.experimental.pallas import tpu as pltpu
from jax.experimental.pallas import tpu_sc as plsc

N = 10000
E = 320000
D_IN = 128
DIM = 32
G = 128

NC = 2
NS = 16
NW = NC * NS
EPW = E // NW
C = 80
K = 128
EPWP = K * C
NBUF = 4
NP = N + 16
NB = 4
BN = NP // NB
RPS = NP // NS

_MESH = dict(core_axis_name="c", subcore_axis_name="s")


def _hist_body(dst_hbm, out_hbm, idx_v, hist_v):
    c = lax.axis_index("c")
    s = lax.axis_index("s")
    w = s * NC + c
    ones = jnp.ones((16,), jnp.float32)
    zeros = jnp.zeros((16,), jnp.float32)
    for e in range(2):
        pltpu.sync_copy(dst_hbm.at[e, w], idx_v)

        def zbody(i, _):
            for u in range(5):
                hist_v[pl.ds((i * 5 + u) * 16, 16)] = zeros
            return 0

        lax.fori_loop(0, N // 80, zbody, 0)

        def body(i, _):
            for u in range(5):
                idx = idx_v[pl.ds((i * 5 + u) * 16, 16)]
                plsc.addupdate_scatter(hist_v, [idx], ones)
            return 0

        lax.fori_loop(0, EPW // 80, body, 0)
        pltpu.sync_copy(hist_v, out_hbm.at[e, w])


def _agg_body(ya_hbm, yb_hbm, ea_hbm, eb_hbm, z_hbm, sa_hbm, sb_hbm,
              idx_sa, idx_da, idx_sb, idx_db, bufs_a, bufs_b,
              acc_a, acc_b, gsem_a, gsem_b, ssem_a, ssem_b):
    c = lax.axis_index("c")
    s = lax.axis_index("s")
    w = s * NC + c
    my_rows = pl.ds(s * RPS, RPS)
    pltpu.sync_copy(z_hbm, acc_a.at[my_rows])
    pltpu.sync_copy(z_hbm, acc_b.at[my_rows])
    pltpu.sync_copy(ea_hbm.at[0, w], idx_sa)
    pltpu.sync_copy(ea_hbm.at[1, w], idx_da)
    pltpu.sync_copy(eb_hbm.at[0, w], idx_sb)
    pltpu.sync_copy(eb_hbm.at[1, w], idx_db)
    plsc.subcore_barrier()

    def gather(j, p):
        pltpu.async_copy(ya_hbm.at[idx_sa.at[j]], bufs_a[p], gsem_a[p])
        pltpu.async_copy(yb_hbm.at[idx_sb.at[j]], bufs_b[p], gsem_b[p])

    def gwait(p):
        pltpu.make_async_copy(ya_hbm.at[idx_sa.at[0]], bufs_a[p],
                              gsem_a[p]).wait()
        pltpu.make_async_copy(yb_hbm.at[idx_sb.at[0]], bufs_b[p],
                              gsem_b[p]).wait()

    def scatter(j, p):
        pltpu.make_async_copy(bufs_a[p], acc_a.at[idx_da.at[j]],
                              ssem_a[p]).start(add=True)
        pltpu.make_async_copy(bufs_b[p], acc_b.at[idx_db.at[j]],
                              ssem_b[p]).start(add=True)

    def swait(p):
        pltpu.make_async_copy(bufs_a[p], acc_a.at[idx_da.at[0]],
                              ssem_a[p]).wait()
        pltpu.make_async_copy(bufs_b[p], acc_b.at[idx_db.at[0]],
                              ssem_b[p]).wait()

    gather(0, 0)
    gather(1, 1)
    gwait(0); scatter(0, 0); gather(2, 2)
    gwait(1); scatter(1, 1); gather(3, 3)

    def steady(jj, _):
        for u in range(NBUF):
            j = NBUF * jj + 2 + u
            p = (2 + u) % NBUF
            pn = u % NBUF
            gwait(p)
            scatter(j, p)
            swait(pn)
            gather(j + 2, pn)
        return 0

    lax.fori_loop(0, (K - 6) // NBUF + 1, steady, 0)
    gwait(2); scatter(K - 2, 2); swait(0)
    gwait(3); scatter(K - 1, 3); swait(1)
    swait(2)
    swait(3)

    plsc.subcore_barrier()
    pltpu.sync_copy(acc_a.at[my_rows], sa_hbm.at[c, my_rows])
    pltpu.sync_copy(acc_b.at[my_rows], sb_hbm.at[c, my_rows])


def _sc_hist(dst2):
    return pl.kernel(
        _hist_body,
        out_type=jax.ShapeDtypeStruct((2, NW, N), jnp.float32),
        mesh=plsc.VectorSubcoreMesh(**_MESH),
        scratch_types=[
            pltpu.VMEM((EPW,), jnp.int32),
            pltpu.VMEM((N,), jnp.float32),
        ],
        compiler_params=pltpu.CompilerParams(needs_layout_passes=False,
                                             use_tc_tiling_on_sc=False),
    )(dst2)


def _sc_agg(ya, yb, ea, eb, zrows):
    return pl.kernel(
        _agg_body,
        out_type=(
            jax.ShapeDtypeStruct((NC, NP, DIM), jnp.float32),
            jax.ShapeDtypeStruct((NC, NP, DIM), jnp.float32),
        ),
        mesh=plsc.VectorSubcoreMesh(**_MESH),
        scratch_types=[
            pltpu.VMEM((K, C), jnp.int32),
            pltpu.VMEM((K, C), jnp.int32),
            pltpu.VMEM((K, C), jnp.int32),
            pltpu.VMEM((K, C), jnp.int32),
            [pltpu.VMEM((C, DIM), jnp.float32) for _ in range(NBUF)],
            [pltpu.VMEM((C, DIM), jnp.float32) for _ in range(NBUF)],
            pltpu.VMEM_SHARED((NP, DIM), jnp.float32),
            pltpu.VMEM_SHARED((NP, DIM), jnp.float32),
            [pltpu.SemaphoreType.DMA for _ in range(NBUF)],
            [pltpu.SemaphoreType.DMA for _ in range(NBUF)],
            [pltpu.SemaphoreType.DMA for _ in range(NBUF)],
            [pltpu.SemaphoreType.DMA for _ in range(NBUF)],
        ],
        compiler_params=pltpu.CompilerParams(needs_layout_passes=False,
                                             use_tc_tiling_on_sc=False),
    )(ya, yb, ea, eb, zrows)


def _mm(a, b):
    return jnp.dot(a, b, preferred_element_type=jnp.float32,
                   precision=lax.Precision.HIGHEST)


def _bs(shape, imap):
    return pl.BlockSpec(shape, imap)


_ROW = lambda i: (i, 0)
_CONST2 = lambda i: (0, 0)
_SROW = lambda i: (0, i, 0)


def _dense1_body(x, w11, w12, dlt, dgt, y1, y2, dil, dig):
    dl = lax.rsqrt(jnp.sum(dlt[...], axis=1, keepdims=True) + 1.0)
    dg = lax.rsqrt(jnp.sum(dgt[...], axis=1, keepdims=True) + 1.0)
    dil[...] = dl
    dig[...] = dg
    xv = x[...]
    y1[...] = _mm(xv, w11[...]) * dl
    y2[...] = _mm(xv, w12[...]) * dg


def _mlp_rows(s1, y1, s2, y2, dl, dg, b1, b2, wa1, wa2, ba, wb, bb):
    x1 = jnp.maximum(dl * (s1[0] + s1[1] + y1) + b1[...], 0.0)
    x2 = jnp.maximum(dg * (s2[0] + s2[1] + y2) + b2[...], 0.0)
    t = jnp.maximum(_mm(x1, wa1[...]) + _mm(x2, wa2[...]) + ba[...], 0.0)
    return _mm(t, wb[...]) + bb[...]


def _dense2_body(s1, y1, s2, y2, dil, dig, b1, b2, wa1, wa2, ba, wb, bb,
                 wc1, wc2, y3, y4):
    dl = dil[...]
    dg = dig[...]
    h = _mlp_rows(s1, y1[...], s2, y2[...], dl, dg,
                  b1, b2, wa1, wa2, ba, wb, bb)
    y3[...] = _mm(h, wc1[...]) * dl
    y4[...] = _mm(h, wc2[...]) * dg


def _dense3_body(s3, y3, s4, y4, dil, dig, b1, b2, wa1, wa2, ba, wb, bb,
                 batch_row, wlin, blin, out, pooled):
    i = pl.program_id(0)
    h = _mlp_rows(s3, y3[...], s4, y4[...], dil[...], dig[...],
                  b1, b2, wa1, wa2, ba, wb, bb)
    rid = lax.broadcasted_iota(jnp.int32, (BN, DIM), 0) + i * BN
    h = jnp.where(rid < N, h, 0.0)
    cid = lax.broadcasted_iota(jnp.int32, (G, BN), 1) + i * BN
    segt = ((batch_row[0] == lax.broadcasted_iota(jnp.int32, (G, BN), 0))
            & (cid < N)).astype(jnp.float32)
    pm = _mm(segt, h)

    @pl.when(i == 0)
    def _():
        pooled[...] = pm

    @pl.when(i != 0)
    def _():
        pooled[...] = pooled[...] + pm

    @pl.when(i == NB - 1)
    def _():
        out[...] = _mm(pooled[...], wlin[...]) + blin[...]


@jax.jit
def kernel(x, edge_index_local, edge_index_global, batch,
           W_c11, b_c11, W_c12, b_c12, W_m1a, b_m1a, W_m1b, b_m1b,
           W_c21, b_c21, W_c22, b_c22, W_m2a, b_m2a, W_m2b, b_m2b,
           W_lin, b_lin):
    f32 = jnp.float32
    dst2 = jnp.stack([edge_index_local[1], edge_index_global[1]]
                     ).reshape(2, NW, EPW)
    pad = ((0, 0), (0, 0), (0, EPWP - EPW))
    ea = jnp.pad(edge_index_local.reshape(2, NW, EPW), pad,
                 constant_values=N).reshape(2, NW, K, C)
    eb = jnp.pad(edge_index_global.reshape(2, NW, EPW), pad,
                 constant_values=N).reshape(2, NW, K, C)
    zrows = jnp.zeros((RPS, DIM), f32)

    deg = _sc_hist(dst2)

    yb_spec = _bs((BN, DIM), _ROW)
    s_spec = _bs((NC, BN, DIM), _SROW)
    d_spec = _bs((BN, 1), _ROW)
    bias_spec = _bs((1, DIM), _CONST2)
    sq_spec = _bs((DIM, DIM), _CONST2)
    mlp_specs = [d_spec, d_spec, bias_spec, bias_spec, sq_spec, sq_spec,
                 bias_spec, sq_spec, bias_spec]

    y1, y2, dil, dig = pl.pallas_call(
        _dense1_body,
        grid=(NB,),
        in_specs=[_bs((BN, D_IN), _ROW),
                  _bs((D_IN, DIM), _CONST2), _bs((D_IN, DIM), _CONST2),
                  _bs((BN, NW), _ROW), _bs((BN, NW), _ROW)],
        out_specs=[yb_spec, yb_spec, d_spec, d_spec],
        out_shape=(
            jax.ShapeDtypeStruct((NP, DIM), f32),
            jax.ShapeDtypeStruct((NP, DIM), f32),
            jax.ShapeDtypeStruct((NP, 1), f32),
            jax.ShapeDtypeStruct((NP, 1), f32),
        ),
    )(jnp.pad(x, ((0, NP - N), (0, 0))), W_c11, W_c12,
      jnp.pad(deg[0].T, ((0, NP - N), (0, 0))),
      jnp.pad(deg[1].T, ((0, NP - N), (0, 0))))

    s1, s2 = _sc_agg(y1, y2, ea, eb, zrows)

    y3, y4 = pl.pallas_call(
        _dense2_body,
        grid=(NB,),
        in_specs=[s_spec, yb_spec, s_spec, yb_spec, *mlp_specs,
                  sq_spec, sq_spec],
        out_specs=[yb_spec, yb_spec],
        out_shape=(jax.ShapeDtypeStruct((NP, DIM), f32),) * 2,
    )(s1, y1, s2, y2, dil, dig,
      b_c11.reshape(1, DIM), b_c12.reshape(1, DIM),
      W_m1a[:DIM], W_m1a[DIM:], b_m1a.reshape(1, DIM),
      W_m1b, b_m1b.reshape(1, DIM), W_c21, W_c22)

    s3, s4 = _sc_agg(y3, y4, ea, eb, zrows)

    out = pl.pallas_call(
        _dense3_body,
        grid=(NB,),
        in_specs=[s_spec, yb_spec, s_spec, yb_spec, *mlp_specs,
                  _bs((1, 1, BN), lambda i: (i, 0, 0)),
                  _bs((DIM, 1), _CONST2), _bs((1, 1), _CONST2)],
        out_specs=pl.BlockSpec((G, 1), lambda i: (0, 0)),
        out_shape=jax.ShapeDtypeStruct((G, 1), f32),
        scratch_shapes=[pltpu.VMEM((G, DIM), f32)],
    )(s3, y3, s4, y4, dil, dig,
      b_c21.reshape(1, DIM), b_c22.reshape(1, DIM),
      W_m2a[:DIM], W_m2a[DIM:], b_m2a.reshape(1, DIM),
      W_m2b, b_m2b.reshape(1, DIM),
      jnp.pad(batch, (0, NP - N), constant_values=-1).reshape(NB, 1, BN),
      W_lin, b_lin.reshape(1, 1))
    return out.reshape(G)


# speedup vs baseline: 1.3300x; 1.3300x over previous
import jax
import jax.numpy as jnp
from jax import lax
from jax.experimental import pallas as pl
from jax.experimental.pallas import tpu as pltpu
from jax.experimental.pallas import tpu_sc as plsc

N = 10000
E = 320000
D_IN = 128
DIM = 32
G = 128

NC = 2
NS = 16
NW = NC * NS
EPW = E // NW
C = 120
K = 84
NG = K
EPWP = K * C
NP = N + 16
YP = N + 8
RPS = NP // NS

_MESH = dict(core_axis_name="c", subcore_axis_name="s")


def _hist_body(dst_hbm, out_hbm, idx_v, hist_v):
    c = lax.axis_index("c")
    s = lax.axis_index("s")
    w = s * NC + c
    ones = jnp.ones((16,), jnp.float32)
    zeros = jnp.zeros((16,), jnp.float32)
    for e in range(2):
        pltpu.sync_copy(dst_hbm.at[e, w], idx_v)

        def zbody(i, _):
            for u in range(5):
                hist_v[pl.ds((i * 5 + u) * 16, 16)] = zeros
            return 0

        lax.fori_loop(0, N // 80, zbody, 0)

        def body(i, _):
            for u in range(5):
                idx = idx_v[pl.ds((i * 5 + u) * 16, 16)]
                plsc.addupdate_scatter(hist_v, [idx], ones)
            return 0

        lax.fori_loop(0, EPW // 80, body, 0)
        pltpu.sync_copy(hist_v, out_hbm.at[e, w])


def _agg_body(ya_hbm, yb_hbm, ea_hbm, eb_hbm, z_hbm, sa_hbm, sb_hbm,
              idx_sa, idx_da, idx_sb, idx_db, bufs_a, bufs_b,
              acc_a, acc_b, gsem_a, gsem_b, ssem_a, ssem_b):
    c = lax.axis_index("c")
    s = lax.axis_index("s")
    w = s * NC + c
    my_rows = pl.ds(s * RPS, RPS)
    pltpu.sync_copy(z_hbm, acc_a.at[my_rows])
    pltpu.sync_copy(z_hbm, acc_b.at[my_rows])
    pltpu.sync_copy(ea_hbm.at[0, w], idx_sa)
    pltpu.sync_copy(ea_hbm.at[1, w], idx_da)
    pltpu.sync_copy(eb_hbm.at[0, w], idx_sb)
    pltpu.sync_copy(eb_hbm.at[1, w], idx_db)
    plsc.subcore_barrier()

    def gather(g, p):
        pltpu.async_copy(ya_hbm.at[idx_sa.at[g]], bufs_a[p], gsem_a[p])
        pltpu.async_copy(yb_hbm.at[idx_sb.at[g]], bufs_b[p], gsem_b[p])

    def gwait(p):
        pltpu.make_async_copy(ya_hbm.at[idx_sa.at[0]], bufs_a[p],
                              gsem_a[p]).wait()
        pltpu.make_async_copy(yb_hbm.at[idx_sb.at[0]], bufs_b[p],
                              gsem_b[p]).wait()

    def scatter(g, p):
        pltpu.make_async_copy(bufs_a[p], acc_a.at[idx_da.at[g]],
                              ssem_a[p]).start(add=True)
        pltpu.make_async_copy(bufs_b[p], acc_b.at[idx_db.at[g]],
                              ssem_b[p]).start(add=True)

    def swait(p):
        pltpu.make_async_copy(bufs_a[p], acc_a.at[idx_da.at[0]],
                              ssem_a[p]).wait()
        pltpu.make_async_copy(bufs_b[p], acc_b.at[idx_db.at[0]],
                              ssem_b[p]).wait()

    gather(0, 0)
    gather(1, 1)
    gwait(0); scatter(0, 0); gather(2, 2)
    gwait(1); scatter(1, 1); gather(3, 3)

    def steady(jj, _):
        for u in range(4):
            g = 4 * jj + 2 + u
            p = (2 + u) % 4
            pn = u % 4
            gwait(p)
            scatter(g, p)
            swait(pn)
            gather(g + 2, pn)
        return 0

    lax.fori_loop(0, (NG - 6) // 4 + 1, steady, 0)
    gwait(2); scatter(NG - 2, 2); swait(0)
    gwait(3); scatter(NG - 1, 3); swait(1)
    swait(2)
    swait(3)

    plsc.subcore_barrier()
    pltpu.sync_copy(acc_a.at[my_rows], sa_hbm.at[c, my_rows])
    pltpu.sync_copy(acc_b.at[my_rows], sb_hbm.at[c, my_rows])


def _sc_hist(dst2):
    return pl.kernel(
        _hist_body,
        out_type=jax.ShapeDtypeStruct((2, NW, N), jnp.float32),
        mesh=plsc.VectorSubcoreMesh(**_MESH),
        scratch_types=[
            pltpu.VMEM((EPW,), jnp.int32),
            pltpu.VMEM((N,), jnp.float32),
        ],
        compiler_params=pltpu.CompilerParams(needs_layout_passes=False,
                                             use_tc_tiling_on_sc=False),
    )(dst2)


def _sc_agg(ya, yb, ea, eb, zrows):
    return pl.kernel(
        _agg_body,
        out_type=(
            jax.ShapeDtypeStruct((NC, NP, DIM), jnp.float32),
            jax.ShapeDtypeStruct((NC, NP, DIM), jnp.float32),
        ),
        mesh=plsc.VectorSubcoreMesh(**_MESH),
        scratch_types=[
            pltpu.VMEM((K, C), jnp.int32),
            pltpu.VMEM((K, C), jnp.int32),
            pltpu.VMEM((K, C), jnp.int32),
            pltpu.VMEM((K, C), jnp.int32),
            [pltpu.VMEM((C, DIM), jnp.float32) for _ in range(4)],
            [pltpu.VMEM((C, DIM), jnp.float32) for _ in range(4)],
            pltpu.VMEM_SHARED((NP, DIM), jnp.float32),
            pltpu.VMEM_SHARED((NP, DIM), jnp.float32),
            [pltpu.SemaphoreType.DMA for _ in range(4)],
            [pltpu.SemaphoreType.DMA for _ in range(4)],
            [pltpu.SemaphoreType.DMA for _ in range(4)],
            [pltpu.SemaphoreType.DMA for _ in range(4)],
        ],
        compiler_params=pltpu.CompilerParams(needs_layout_passes=False,
                                             use_tc_tiling_on_sc=False),
    )(ya, yb, ea, eb, zrows)


def _mm(a, b):
    return jnp.dot(a, b, preferred_element_type=jnp.float32,
                   precision=lax.Precision.HIGHEST)


def _dense1_body(xt, w11t, w12t, dlp, dgp, y1t, y2t, dil, dig):
    dl = lax.rsqrt(jnp.sum(dlp[...], axis=0, keepdims=True) + 1.0)
    dg = lax.rsqrt(jnp.sum(dgp[...], axis=0, keepdims=True) + 1.0)
    dil[...] = dl
    dig[...] = dg
    xv = xt[...]
    y1t[...] = _mm(w11t[...], xv) * dl
    y2t[...] = _mm(w12t[...], xv) * dg


def _mlp_t(s1, y1, s2, y2, dl, dg, b1, b2, wa1, wa2, ba, wb, bb):
    x1 = jnp.maximum(dl * (s1[0] + s1[1] + y1) + b1[...], 0.0)
    x2 = jnp.maximum(dg * (s2[0] + s2[1] + y2) + b2[...], 0.0)
    t = jnp.maximum(_mm(wa1[...], x1) + _mm(wa2[...], x2) + ba[...], 0.0)
    return _mm(wb[...], t) + bb[...]


def _dense2_body(s1, y1, s2, y2, dil, dig, b1, b2, wa1, wa2, ba, wb, bb,
                 wc1, wc2, y3t, y4t):
    dl = dil[...]
    dg = dig[...]
    h = _mlp_t(s1, y1[...], s2, y2[...], dl, dg, b1, b2, wa1, wa2, ba, wb, bb)
    y3t[...] = _mm(wc1[...], h) * dl
    y4t[...] = _mm(wc2[...], h) * dg


def _dense3_body(s3, y3, s4, y4, dil, dig, b1, b2, wa1, wa2, ba, wb, bb,
                 batch_col, wlint, blin, out):
    h = _mlp_t(s3, y3[...], s4, y4[...], dil[...], dig[...],
               b1, b2, wa1, wa2, ba, wb, bb)
    seg = (batch_col[...] == lax.broadcasted_iota(jnp.int32, (N, G), 1))
    pooled = _mm(h, seg.astype(jnp.float32))
    out[...] = _mm(wlint[...], pooled) + blin[...]


def _tc(body, out_shapes):
    return pl.pallas_call(body, out_shape=out_shapes)


@jax.jit
def kernel(x, edge_index_local, edge_index_global, batch,
           W_c11, b_c11, W_c12, b_c12, W_m1a, b_m1a, W_m1b, b_m1b,
           W_c21, b_c21, W_c22, b_c22, W_m2a, b_m2a, W_m2b, b_m2b,
           W_lin, b_lin):
    f32 = jnp.float32
    dst2 = jnp.stack([edge_index_local[1], edge_index_global[1]]
                     ).reshape(2, NW, EPW)
    pad = ((0, 0), (0, 0), (0, EPWP - EPW))
    ea = jnp.pad(edge_index_local.reshape(2, NW, EPW), pad,
                 constant_values=N).reshape(2, NW, K, C)
    eb = jnp.pad(edge_index_global.reshape(2, NW, EPW), pad,
                 constant_values=N).reshape(2, NW, K, C)
    zrows = jnp.zeros((RPS, DIM), f32)

    deg = _sc_hist(dst2)

    y1t, y2t, dil, dig = _tc(_dense1_body, (
        jax.ShapeDtypeStruct((DIM, N), f32),
        jax.ShapeDtypeStruct((DIM, N), f32),
        jax.ShapeDtypeStruct((1, N), f32),
        jax.ShapeDtypeStruct((1, N), f32),
    ))(x.T, W_c11.T, W_c12.T, deg[0], deg[1])

    ypad = ((0, YP - N), (0, 0))
    y1, y2 = jnp.pad(y1t.T, ypad), jnp.pad(y2t.T, ypad)
    s1, s2 = _sc_agg(y1, y2, ea, eb, zrows)

    yy = (jax.ShapeDtypeStruct((DIM, N), f32),) * 2
    y3t, y4t = _tc(_dense2_body, yy)(
        s1[:, :N].transpose(0, 2, 1), y1t,
        s2[:, :N].transpose(0, 2, 1), y2t, dil, dig,
        b_c11.reshape(DIM, 1), b_c12.reshape(DIM, 1),
        W_m1a[:DIM].T, W_m1a[DIM:].T, b_m1a.reshape(DIM, 1),
        W_m1b.T, b_m1b.reshape(DIM, 1), W_c21.T, W_c22.T)

    s3, s4 = _sc_agg(jnp.pad(y3t.T, ypad), jnp.pad(y4t.T, ypad),
                     ea, eb, zrows)

    out = _tc(_dense3_body, jax.ShapeDtypeStruct((1, G), f32))(
        s3[:, :N].transpose(0, 2, 1), y3t,
        s4[:, :N].transpose(0, 2, 1), y4t, dil, dig,
        b_c21.reshape(DIM, 1), b_c22.reshape(DIM, 1),
        W_m2a[:DIM].T, W_m2a[DIM:].T, b_m2a.reshape(DIM, 1),
        W_m2b.T, b_m2b.reshape(DIM, 1),
        batch.reshape(N, 1), W_lin.T, b_lin.reshape(1, 1))
    return out.reshape(G)
